# baseline (speedup 1.0000x reference)
import functools

import jax
import jax.numpy as jnp
from jax.experimental import pallas as pl
from jax.experimental.pallas import tpu as pltpu

K = 7
P = 3
LANES = 128


def _fold_flat(t, f, n_blocks, hw):
    while n_blocks > 2:
        half_rows = (n_blocks // 2) * hw // LANES
        t = f(t[:half_rows], t[half_rows:])
        n_blocks //= 2
    rows_top = (hw + LANES - 1) // LANES
    off = hw % LANES
    if off == 0:
        return f(t[:rows_top], t[rows_top:])
    start = hw // LANES
    a1 = pltpu.roll(t[start:start + rows_top], off, axis=1)
    a1_up = jnp.concatenate(
        [a1[1:], jnp.zeros((1, LANES), jnp.float32)], axis=0)
    lane = jax.lax.broadcasted_iota(jnp.int32, (rows_top, LANES), 1)
    b = jnp.where(lane < LANES - off, a1, a1_up)
    return f(t[:rows_top], b)


def _store_flat(o_ref, val, hw):
    n_full = hw // LANES
    for r in range(n_full):
        o_ref[0, 0:1, r * LANES:(r + 1) * LANES] = val[r:r + 1]
    rem = hw - n_full * LANES
    if rem:
        o_ref[0, 0:1, n_full * LANES:hw] = val[n_full:n_full + 1, 0:rem]


def _reduce_packed_body(x_ref, avg_ref, max_ref, *, groups, c, hw, inv_c):
    v = x_ref[0]
    rows = v.shape[0]
    v3 = v.reshape(groups, rows // groups, LANES)
    s = jnp.sum(v3, axis=0)
    m = jnp.max(v3, axis=0)
    n_blocks = c // groups
    fs = _fold_flat(s, jnp.add, n_blocks, hw)
    fm = _fold_flat(m, jnp.maximum, n_blocks, hw)
    _store_flat(avg_ref, fs * inv_c, hw)
    _store_flat(max_ref, fm, hw)


def _reduce_flat_body(x_ref, avg_ref, max_ref, *, inv_c):
    v = x_ref[...]
    avg_ref[0] = jnp.sum(v, axis=1) * inv_c
    max_ref[0] = jnp.max(v, axis=1)


def _conv_body(avg_ref, max_ref, s_ref, o_ref, pad_ref, *, nb, h, w):
    hp, wp = h + 2 * P, w + 2 * P
    pad_ref[...] = jnp.zeros_like(pad_ref)
    pad_ref[:, P:P + h, P:P + w] = avg_ref[...]
    pad_ref[:, P:P + h, wp + P:wp + P + w] = max_ref[...]
    s = s_ref[...]
    for b in range(nb):
        t = jnp.dot(pad_ref[b], s, preferred_element_type=jnp.float32)
        acc = t[0:h, 0:w]
        for dy in range(1, K):
            acc = acc + t[dy:dy + h, dy * w:dy * w + w]
        o_ref[b] = jax.nn.sigmoid(acc)


def _band_matrix(conv_weight, w, wp):
    wm = conv_weight.reshape(2, K, K).astype(jnp.float32)
    ci = jnp.arange(wp)
    c = jnp.arange(w)
    dx = ci[:, None] - c[None, :]
    valid = (dx >= 0) & (dx < K)
    dxc = jnp.clip(dx, 0, K - 1)
    g = wm[:, :, dxc]
    g = jnp.where(valid[None, None], g, 0.0)
    return g.transpose(0, 2, 1, 3).reshape(2 * wp, K * w)


def _channel_reduce(x):
    N, C, H, W = x.shape
    HW = H * W
    CHW = C * HW
    groups = 16
    packed_ok = (
        C % groups == 0
        and CHW % (groups * LANES) == 0
        and (CHW // (groups * LANES)) % 8 == 0
        and (C // groups) & (C // groups - 1) == 0
        and 2 * HW % LANES == 0
        and HW % LANES in (0, LANES // 2)
    )
    out_shape = (jax.ShapeDtypeStruct((N, 1, HW), jnp.float32),
                 jax.ShapeDtypeStruct((N, 1, HW), jnp.float32))
    out_specs = (pl.BlockSpec((1, 1, HW), lambda i: (i, 0, 0)),
                 pl.BlockSpec((1, 1, HW), lambda i: (i, 0, 0)))
    if packed_ok:
        rows = CHW // LANES
        return pl.pallas_call(
            functools.partial(_reduce_packed_body, groups=groups, c=C,
                              hw=HW, inv_c=1.0 / C),
            out_shape=out_shape,
            grid=(N,),
            in_specs=[pl.BlockSpec((1, rows, LANES), lambda i: (i, 0, 0))],
            out_specs=out_specs,
            compiler_params=pltpu.CompilerParams(
                dimension_semantics=("parallel",),
                vmem_limit_bytes=48 << 20),
            cost_estimate=pl.CostEstimate(
                flops=2 * N * CHW,
                transcendentals=0,
                bytes_accessed=(N * CHW + 2 * N * HW) * 4),
        )(x.reshape(N, rows, LANES))
    return pl.pallas_call(
        functools.partial(_reduce_flat_body, inv_c=1.0 / C),
        out_shape=out_shape,
        grid=(N,),
        in_specs=[pl.BlockSpec((1, C, HW), lambda i: (i, 0, 0))],
        out_specs=out_specs,
        compiler_params=pltpu.CompilerParams(
            dimension_semantics=("parallel",),
            vmem_limit_bytes=48 << 20),
        cost_estimate=pl.CostEstimate(
            flops=2 * N * CHW,
            transcendentals=0,
            bytes_accessed=(N * CHW + 2 * N * HW) * 4),
    )(x.reshape(N, C, HW))


def kernel(x, conv_weight):
    N, C, H, W = x.shape
    HW = H * W
    Hp, Wp = H + 2 * P, W + 2 * P
    nbc = 8 if N % 8 == 0 else 1

    avg, mx = _channel_reduce(x)
    s_mat = _band_matrix(conv_weight, W, Wp)

    out = pl.pallas_call(
        functools.partial(_conv_body, nb=nbc, h=H, w=W),
        out_shape=jax.ShapeDtypeStruct((N, H, W), x.dtype),
        grid=(N // nbc,),
        in_specs=[
            pl.BlockSpec((nbc, H, W), lambda i: (i, 0, 0)),
            pl.BlockSpec((nbc, H, W), lambda i: (i, 0, 0)),
            pl.BlockSpec((2 * Wp, K * W), lambda i: (0, 0)),
        ],
        out_specs=pl.BlockSpec((nbc, H, W), lambda i: (i, 0, 0)),
        scratch_shapes=[pltpu.VMEM((nbc, Hp, 2 * Wp), jnp.float32)],
        compiler_params=pltpu.CompilerParams(
            dimension_semantics=("parallel",),
            vmem_limit_bytes=32 << 20),
        cost_estimate=pl.CostEstimate(
            flops=2 * N * Hp * 2 * Wp * K * W + 8 * N * HW,
            transcendentals=N * HW,
            bytes_accessed=(3 * N * HW + 2 * Wp * K * W) * 4),
    )(avg.reshape(N, H, W), mx.reshape(N, H, W), s_mat)

    return out.reshape(N, 1, H, W)


# speedup vs baseline: 5.3784x; 5.3784x over previous
import functools

import jax
import jax.numpy as jnp
from jax.experimental import pallas as pl
from jax.experimental.pallas import tpu as pltpu

K = 7
P = 3
LANES = 128


def _reduce_nhwc_body(x_ref, avg_ref, max_ref, *, c, inv_c):
    v = x_ref[0]
    half = c // 2
    s = v[:, :, :half] + v[:, :, half:]
    m = jnp.maximum(v[:, :, :half], v[:, :, half:])
    while half > LANES:
        half //= 2
        s = s[:, :, :half] + s[:, :, half:]
        m = jnp.maximum(m[:, :, :half], m[:, :, half:])
    avg_ref[0] = jnp.sum(s, axis=-1) * inv_c
    max_ref[0] = jnp.max(m, axis=-1)


def _reduce_flat_body(x_ref, avg_ref, max_ref, *, inv_c):
    v = x_ref[...]
    avg_ref[0] = jnp.sum(v, axis=1) * inv_c
    max_ref[0] = jnp.max(v, axis=1)


def _conv_body(avg_ref, max_ref, s_ref, o_ref, pad_ref, *, nb, h, w):
    hp, wp = h + 2 * P, w + 2 * P
    pad_ref[...] = jnp.zeros_like(pad_ref)
    pad_ref[:, P:P + h, P:P + w] = avg_ref[...]
    pad_ref[:, P:P + h, wp + P:wp + P + w] = max_ref[...]
    s = s_ref[...]
    for b in range(nb):
        t = jnp.dot(pad_ref[b], s, preferred_element_type=jnp.float32)
        acc = t[0:h, 0:w]
        for dy in range(1, K):
            acc = acc + t[dy:dy + h, dy * w:dy * w + w]
        o_ref[b] = jax.nn.sigmoid(acc)


def _band_matrix(conv_weight, w, wp):
    wm = conv_weight.reshape(2, K, K).astype(jnp.float32)
    ci = jnp.arange(wp)
    c = jnp.arange(w)
    dx = ci[:, None] - c[None, :]
    valid = (dx >= 0) & (dx < K)
    dxc = jnp.clip(dx, 0, K - 1)
    g = wm[:, :, dxc]
    g = jnp.where(valid[None, None], g, 0.0)
    return g.transpose(0, 2, 1, 3).reshape(2 * wp, K * w)


def _channel_reduce(x):
    N, C, H, W = x.shape
    HW = H * W
    out_shape = (jax.ShapeDtypeStruct((N, H, W), jnp.float32),
                 jax.ShapeDtypeStruct((N, H, W), jnp.float32))
    out_specs = (pl.BlockSpec((1, H, W), lambda i: (i, 0, 0)),
                 pl.BlockSpec((1, H, W), lambda i: (i, 0, 0)))
    lanes_ok = C % LANES == 0 and ((C // LANES) & (C // LANES - 1)) == 0
    if lanes_ok:
        xt = jnp.transpose(x, (0, 2, 3, 1))
        return pl.pallas_call(
            functools.partial(_reduce_nhwc_body, c=C, inv_c=1.0 / C),
            out_shape=out_shape,
            grid=(N,),
            in_specs=[pl.BlockSpec((1, H, W, C), lambda i: (i, 0, 0, 0))],
            out_specs=out_specs,
            compiler_params=pltpu.CompilerParams(
                dimension_semantics=("parallel",),
                vmem_limit_bytes=48 << 20),
            cost_estimate=pl.CostEstimate(
                flops=2 * N * C * HW,
                transcendentals=0,
                bytes_accessed=(N * C * HW + 2 * N * HW) * 4),
        )(xt)
    avg, mx = pl.pallas_call(
        functools.partial(_reduce_flat_body, inv_c=1.0 / C),
        out_shape=(jax.ShapeDtypeStruct((N, 1, HW), jnp.float32),
                   jax.ShapeDtypeStruct((N, 1, HW), jnp.float32)),
        grid=(N,),
        in_specs=[pl.BlockSpec((1, C, HW), lambda i: (i, 0, 0))],
        out_specs=(pl.BlockSpec((1, 1, HW), lambda i: (i, 0, 0)),
                   pl.BlockSpec((1, 1, HW), lambda i: (i, 0, 0))),
        compiler_params=pltpu.CompilerParams(
            dimension_semantics=("parallel",),
            vmem_limit_bytes=48 << 20),
        cost_estimate=pl.CostEstimate(
            flops=2 * N * C * HW,
            transcendentals=0,
            bytes_accessed=(N * C * HW + 2 * N * HW) * 4),
    )(x.reshape(N, C, HW))
    return avg.reshape(N, H, W), mx.reshape(N, H, W)


def kernel(x, conv_weight):
    N, C, H, W = x.shape
    HW = H * W
    Hp, Wp = H + 2 * P, W + 2 * P
    nbc = 8 if N % 8 == 0 else 1

    avg, mx = _channel_reduce(x)
    s_mat = _band_matrix(conv_weight, W, Wp)

    out = pl.pallas_call(
        functools.partial(_conv_body, nb=nbc, h=H, w=W),
        out_shape=jax.ShapeDtypeStruct((N, H, W), x.dtype),
        grid=(N // nbc,),
        in_specs=[
            pl.BlockSpec((nbc, H, W), lambda i: (i, 0, 0)),
            pl.BlockSpec((nbc, H, W), lambda i: (i, 0, 0)),
            pl.BlockSpec((2 * Wp, K * W), lambda i: (0, 0)),
        ],
        out_specs=pl.BlockSpec((nbc, H, W), lambda i: (i, 0, 0)),
        scratch_shapes=[pltpu.VMEM((nbc, Hp, 2 * Wp), jnp.float32)],
        compiler_params=pltpu.CompilerParams(
            dimension_semantics=("parallel",),
            vmem_limit_bytes=32 << 20),
        cost_estimate=pl.CostEstimate(
            flops=2 * N * Hp * 2 * Wp * K * W + 8 * N * HW,
            transcendentals=N * HW,
            bytes_accessed=(3 * N * HW + 2 * Wp * K * W) * 4),
    )(avg, mx, s_mat)

    return out.reshape(N, 1, H, W)


# speedup vs baseline: 6.4384x; 1.1971x over previous
import functools

import jax
import jax.numpy as jnp
from jax.experimental import pallas as pl
from jax.experimental.pallas import tpu as pltpu

K = 7
P = 3
LANES = 128


def _reduce_nhwc_body(x_ref, avg_ref, max_ref, *, nb, c, inv_c):
    for b in range(nb):
        v = x_ref[b]
        half = c // 2
        s = v[:, :, :half] + v[:, :, half:]
        m = jnp.maximum(v[:, :, :half], v[:, :, half:])
        while half > LANES:
            half //= 2
            s = s[:, :, :half] + s[:, :, half:]
            m = jnp.maximum(m[:, :, :half], m[:, :, half:])
        avg_ref[b] = jnp.sum(s, axis=-1) * inv_c
        max_ref[b] = jnp.max(m, axis=-1)


def _reduce_flat_body(x_ref, avg_ref, max_ref, *, inv_c):
    v = x_ref[...]
    avg_ref[0] = jnp.sum(v, axis=1) * inv_c
    max_ref[0] = jnp.max(v, axis=1)


def _conv_body(avg_ref, max_ref, s_ref, o_ref, pad_ref, *, nb, h, w):
    hp, wp = h + 2 * P, w + 2 * P
    pad_ref[...] = jnp.zeros_like(pad_ref)
    pad_ref[:, P:P + h, P:P + w] = avg_ref[...]
    pad_ref[:, P:P + h, wp + P:wp + P + w] = max_ref[...]
    s = s_ref[...]
    for b in range(nb):
        t = jnp.dot(pad_ref[b], s, preferred_element_type=jnp.float32)
        acc = t[0:h, 0:w]
        for dy in range(1, K):
            acc = acc + t[dy:dy + h, dy * w:dy * w + w]
        o_ref[b] = jax.nn.sigmoid(acc)


def _band_matrix(conv_weight, w, wp):
    wm = conv_weight.reshape(2, K, K).astype(jnp.float32)
    ci = jnp.arange(wp)
    c = jnp.arange(w)
    dx = ci[:, None] - c[None, :]
    valid = (dx >= 0) & (dx < K)
    dxc = jnp.clip(dx, 0, K - 1)
    g = wm[:, :, dxc]
    g = jnp.where(valid[None, None], g, 0.0)
    return g.transpose(0, 2, 1, 3).reshape(2 * wp, K * w)


def _channel_reduce(x):
    N, C, H, W = x.shape
    HW = H * W
    out_shape = (jax.ShapeDtypeStruct((N, H, W), jnp.float32),
                 jax.ShapeDtypeStruct((N, H, W), jnp.float32))
    out_specs = (pl.BlockSpec((1, H, W), lambda i: (i, 0, 0)),
                 pl.BlockSpec((1, H, W), lambda i: (i, 0, 0)))
    lanes_ok = C % LANES == 0 and ((C // LANES) & (C // LANES - 1)) == 0
    if lanes_ok:
        nb = 4 if N % 4 == 0 else 1
        xt = jnp.transpose(x, (0, 2, 3, 1))
        return pl.pallas_call(
            functools.partial(_reduce_nhwc_body, nb=nb, c=C, inv_c=1.0 / C),
            out_shape=out_shape,
            grid=(N // nb,),
            in_specs=[pl.BlockSpec((nb, H, W, C), lambda i: (i, 0, 0, 0))],
            out_specs=(pl.BlockSpec((nb, H, W), lambda i: (i, 0, 0)),
                       pl.BlockSpec((nb, H, W), lambda i: (i, 0, 0))),
            compiler_params=pltpu.CompilerParams(
                dimension_semantics=("parallel",),
                vmem_limit_bytes=48 << 20),
            cost_estimate=pl.CostEstimate(
                flops=2 * N * C * HW,
                transcendentals=0,
                bytes_accessed=(N * C * HW + 2 * N * HW) * 4),
        )(xt)
    avg, mx = pl.pallas_call(
        functools.partial(_reduce_flat_body, inv_c=1.0 / C),
        out_shape=(jax.ShapeDtypeStruct((N, 1, HW), jnp.float32),
                   jax.ShapeDtypeStruct((N, 1, HW), jnp.float32)),
        grid=(N,),
        in_specs=[pl.BlockSpec((1, C, HW), lambda i: (i, 0, 0))],
        out_specs=(pl.BlockSpec((1, 1, HW), lambda i: (i, 0, 0)),
                   pl.BlockSpec((1, 1, HW), lambda i: (i, 0, 0))),
        compiler_params=pltpu.CompilerParams(
            dimension_semantics=("parallel",),
            vmem_limit_bytes=48 << 20),
        cost_estimate=pl.CostEstimate(
            flops=2 * N * C * HW,
            transcendentals=0,
            bytes_accessed=(N * C * HW + 2 * N * HW) * 4),
    )(x.reshape(N, C, HW))
    return avg.reshape(N, H, W), mx.reshape(N, H, W)


def kernel(x, conv_weight):
    N, C, H, W = x.shape
    HW = H * W
    Hp, Wp = H + 2 * P, W + 2 * P
    nbc = 8 if N % 8 == 0 else 1

    avg, mx = _channel_reduce(x)
    s_mat = _band_matrix(conv_weight, W, Wp)

    out = pl.pallas_call(
        functools.partial(_conv_body, nb=nbc, h=H, w=W),
        out_shape=jax.ShapeDtypeStruct((N, H, W), x.dtype),
        grid=(N // nbc,),
        in_specs=[
            pl.BlockSpec((nbc, H, W), lambda i: (i, 0, 0)),
            pl.BlockSpec((nbc, H, W), lambda i: (i, 0, 0)),
            pl.BlockSpec((2 * Wp, K * W), lambda i: (0, 0)),
        ],
        out_specs=pl.BlockSpec((nbc, H, W), lambda i: (i, 0, 0)),
        scratch_shapes=[pltpu.VMEM((nbc, Hp, 2 * Wp), jnp.float32)],
        compiler_params=pltpu.CompilerParams(
            dimension_semantics=("parallel",),
            vmem_limit_bytes=32 << 20),
        cost_estimate=pl.CostEstimate(
            flops=2 * N * Hp * 2 * Wp * K * W + 8 * N * HW,
            transcendentals=N * HW,
            bytes_accessed=(3 * N * HW + 2 * Wp * K * W) * 4),
    )(avg, mx, s_mat)

    return out.reshape(N, 1, H, W)


# speedup vs baseline: 6.9396x; 1.0778x over previous
import functools

import jax
import jax.numpy as jnp
from jax.experimental import pallas as pl
from jax.experimental.pallas import tpu as pltpu

K = 7
P = 3
LANES = 128


def _reduce_lanes(v, c, inv_c):
    half = c // 2
    s = v[:, :, :half] + v[:, :, half:]
    m = jnp.maximum(v[:, :, :half], v[:, :, half:])
    while half > LANES:
        half //= 2
        s = s[:, :, :half] + s[:, :, half:]
        m = jnp.maximum(m[:, :, :half], m[:, :, half:])
    return jnp.sum(s, axis=-1) * inv_c, jnp.max(m, axis=-1)


def _conv_from_pad(pad_b, s, h, w):
    t = jnp.dot(pad_b, s, preferred_element_type=jnp.float32)
    acc = t[0:h, 0:w]
    for dy in range(1, K):
        acc = acc + t[dy:dy + h, dy * w:dy * w + w]
    return jax.nn.sigmoid(acc)


def _fused_body(s_ref, x_ref, o_ref, pad_ref, *, nb, c, h, w, inv_c):
    wp = w + 2 * P
    pad_ref[...] = jnp.zeros_like(pad_ref)
    for b in range(nb):
        avg, mx = _reduce_lanes(x_ref[b], c, inv_c)
        pad_ref[b, P:P + h, P:P + w] = avg
        pad_ref[b, P:P + h, wp + P:wp + P + w] = mx
    s = s_ref[...]
    for b in range(nb):
        o_ref[b] = _conv_from_pad(pad_ref[b], s, h, w)


def _reduce_flat_body(x_ref, avg_ref, max_ref, *, inv_c):
    v = x_ref[...]
    avg_ref[0] = jnp.sum(v, axis=1) * inv_c
    max_ref[0] = jnp.max(v, axis=1)


def _conv_body(avg_ref, max_ref, s_ref, o_ref, pad_ref, *, nb, h, w):
    wp = w + 2 * P
    pad_ref[...] = jnp.zeros_like(pad_ref)
    pad_ref[:, P:P + h, P:P + w] = avg_ref[...]
    pad_ref[:, P:P + h, wp + P:wp + P + w] = max_ref[...]
    s = s_ref[...]
    for b in range(nb):
        o_ref[b] = _conv_from_pad(pad_ref[b], s, h, w)


def _band_matrix(conv_weight, w, wp):
    wm = conv_weight.reshape(2, K, K).astype(jnp.float32)
    ci = jnp.arange(wp)
    c = jnp.arange(w)
    dx = ci[:, None] - c[None, :]
    valid = (dx >= 0) & (dx < K)
    dxc = jnp.clip(dx, 0, K - 1)
    g = wm[:, :, dxc]
    g = jnp.where(valid[None, None], g, 0.0)
    return g.transpose(0, 2, 1, 3).reshape(2 * wp, K * w)


def _generic_path(x, s_mat):
    N, C, H, W = x.shape
    HW = H * W
    Hp, Wp = H + 2 * P, W + 2 * P
    nbc = 8 if N % 8 == 0 else 1
    avg, mx = pl.pallas_call(
        functools.partial(_reduce_flat_body, inv_c=1.0 / C),
        out_shape=(jax.ShapeDtypeStruct((N, 1, HW), jnp.float32),
                   jax.ShapeDtypeStruct((N, 1, HW), jnp.float32)),
        grid=(N,),
        in_specs=[pl.BlockSpec((1, C, HW), lambda i: (i, 0, 0))],
        out_specs=(pl.BlockSpec((1, 1, HW), lambda i: (i, 0, 0)),
                   pl.BlockSpec((1, 1, HW), lambda i: (i, 0, 0))),
        compiler_params=pltpu.CompilerParams(
            dimension_semantics=("parallel",),
            vmem_limit_bytes=48 << 20),
        cost_estimate=pl.CostEstimate(
            flops=2 * N * C * HW, transcendentals=0,
            bytes_accessed=(N * C * HW + 2 * N * HW) * 4),
    )(x.reshape(N, C, HW))
    return pl.pallas_call(
        functools.partial(_conv_body, nb=nbc, h=H, w=W),
        out_shape=jax.ShapeDtypeStruct((N, H, W), x.dtype),
        grid=(N // nbc,),
        in_specs=[
            pl.BlockSpec((nbc, H, W), lambda i: (i, 0, 0)),
            pl.BlockSpec((nbc, H, W), lambda i: (i, 0, 0)),
            pl.BlockSpec((2 * Wp, K * W), lambda i: (0, 0)),
        ],
        out_specs=pl.BlockSpec((nbc, H, W), lambda i: (i, 0, 0)),
        scratch_shapes=[pltpu.VMEM((nbc, Hp, 2 * Wp), jnp.float32)],
        compiler_params=pltpu.CompilerParams(
            dimension_semantics=("parallel",),
            vmem_limit_bytes=32 << 20),
        cost_estimate=pl.CostEstimate(
            flops=2 * N * Hp * 2 * Wp * K * W + 8 * N * HW,
            transcendentals=N * HW,
            bytes_accessed=(3 * N * HW + 2 * Wp * K * W) * 4),
    )(avg.reshape(N, H, W), mx.reshape(N, H, W), s_mat)


def kernel(x, conv_weight):
    N, C, H, W = x.shape
    HW = H * W
    Hp, Wp = H + 2 * P, W + 2 * P

    s_mat = _band_matrix(conv_weight, W, Wp)
    lanes_ok = C % LANES == 0 and ((C // LANES) & (C // LANES - 1)) == 0
    if not lanes_ok:
        return _generic_path(x, s_mat).reshape(N, 1, H, W)

    nb = 4 if N % 4 == 0 else 1
    xt = jnp.transpose(x, (0, 2, 3, 1))
    out = pl.pallas_call(
        functools.partial(_fused_body, nb=nb, c=C, h=H, w=W, inv_c=1.0 / C),
        out_shape=jax.ShapeDtypeStruct((N, H, W), x.dtype),
        grid=(N // nb,),
        in_specs=[
            pl.BlockSpec((2 * Wp, K * W), lambda i: (0, 0)),
            pl.BlockSpec((nb, H, W, C), lambda i: (i, 0, 0, 0)),
        ],
        out_specs=pl.BlockSpec((nb, H, W), lambda i: (i, 0, 0)),
        scratch_shapes=[pltpu.VMEM((nb, Hp, 2 * Wp), jnp.float32)],
        compiler_params=pltpu.CompilerParams(
            dimension_semantics=("parallel",),
            vmem_limit_bytes=48 << 20),
        cost_estimate=pl.CostEstimate(
            flops=2 * N * C * HW + 2 * N * Hp * 2 * Wp * K * W,
            transcendentals=N * HW,
            bytes_accessed=(N * C * HW + N * HW) * 4),
    )(s_mat, xt)
    return out.reshape(N, 1, H, W)


# speedup vs baseline: 8.2807x; 1.1933x over previous
import functools

import jax
import jax.numpy as jnp
from jax.experimental import pallas as pl
from jax.experimental.pallas import tpu as pltpu

K = 7
P = 3
LANES = 128


def _reduce_lanes(v, c, inv_c):
    half = c // 2
    s = v[:, :, :half] + v[:, :, half:]
    m = jnp.maximum(v[:, :, :half], v[:, :, half:])
    while half > LANES:
        half //= 2
        s = s[:, :, :half] + s[:, :, half:]
        m = jnp.maximum(m[:, :, :half], m[:, :, half:])
    return jnp.sum(s, axis=-1) * inv_c, jnp.max(m, axis=-1)


def _conv_from_pad(pad_b, s, h, w):
    t = jnp.dot(pad_b, s, preferred_element_type=jnp.float32)
    acc = t[0:h, 0:w]
    for dy in range(1, K):
        acc = acc + t[dy:dy + h, dy * w:dy * w + w]
    return jax.nn.sigmoid(acc)


def _fused_body(s_ref, x_ref, o_ref, pad_ref, *, nb, c, h, w, inv_c):
    wp = w + 2 * P
    pad_ref[...] = jnp.zeros_like(pad_ref)
    for b in range(nb):
        avg, mx = _reduce_lanes(x_ref[b], c, inv_c)
        pad_ref[b, P:P + h, P:P + w] = avg
        pad_ref[b, P:P + h, wp + P:wp + P + w] = mx
    s = s_ref[...]
    for b in range(nb):
        o_ref[b] = _conv_from_pad(pad_ref[b], s, h, w)


def _reduce_flat_body(x_ref, avg_ref, max_ref, *, inv_c):
    v = x_ref[...]
    avg_ref[0] = jnp.sum(v, axis=1) * inv_c
    max_ref[0] = jnp.max(v, axis=1)


def _conv_body(avg_ref, max_ref, s_ref, o_ref, pad_ref, *, nb, h, w):
    wp = w + 2 * P
    pad_ref[...] = jnp.zeros_like(pad_ref)
    pad_ref[:, P:P + h, P:P + w] = avg_ref[...]
    pad_ref[:, P:P + h, wp + P:wp + P + w] = max_ref[...]
    s = s_ref[...]
    for b in range(nb):
        o_ref[b] = _conv_from_pad(pad_ref[b], s, h, w)


def _band_matrix(conv_weight, w, wp):
    wm = conv_weight.reshape(2, K, K).astype(jnp.float32)
    i = jnp.arange(2 * wp)[:, None]
    j = jnp.arange(K * w)[None, :]
    dx = (i % wp) - (j % w)
    s = jnp.zeros((2 * wp, K * w), jnp.float32)
    for k in range(K):
        wk = jnp.broadcast_to(wm[:, None, :, None][:, :, :, :, k],
                              (2, wp, K, w)).reshape(2 * wp, K * w)
        s = s + jnp.where(dx == k, wk, 0.0)
    return s


def _generic_path(x, s_mat):
    N, C, H, W = x.shape
    HW = H * W
    Hp, Wp = H + 2 * P, W + 2 * P
    nbc = 8 if N % 8 == 0 else 1
    avg, mx = pl.pallas_call(
        functools.partial(_reduce_flat_body, inv_c=1.0 / C),
        out_shape=(jax.ShapeDtypeStruct((N, 1, HW), jnp.float32),
                   jax.ShapeDtypeStruct((N, 1, HW), jnp.float32)),
        grid=(N,),
        in_specs=[pl.BlockSpec((1, C, HW), lambda i: (i, 0, 0))],
        out_specs=(pl.BlockSpec((1, 1, HW), lambda i: (i, 0, 0)),
                   pl.BlockSpec((1, 1, HW), lambda i: (i, 0, 0))),
        compiler_params=pltpu.CompilerParams(
            dimension_semantics=("parallel",),
            vmem_limit_bytes=48 << 20),
        cost_estimate=pl.CostEstimate(
            flops=2 * N * C * HW, transcendentals=0,
            bytes_accessed=(N * C * HW + 2 * N * HW) * 4),
    )(x.reshape(N, C, HW))
    return pl.pallas_call(
        functools.partial(_conv_body, nb=nbc, h=H, w=W),
        out_shape=jax.ShapeDtypeStruct((N, H, W), x.dtype),
        grid=(N // nbc,),
        in_specs=[
            pl.BlockSpec((nbc, H, W), lambda i: (i, 0, 0)),
            pl.BlockSpec((nbc, H, W), lambda i: (i, 0, 0)),
            pl.BlockSpec((2 * Wp, K * W), lambda i: (0, 0)),
        ],
        out_specs=pl.BlockSpec((nbc, H, W), lambda i: (i, 0, 0)),
        scratch_shapes=[pltpu.VMEM((nbc, Hp, 2 * Wp), jnp.float32)],
        compiler_params=pltpu.CompilerParams(
            dimension_semantics=("parallel",),
            vmem_limit_bytes=32 << 20),
        cost_estimate=pl.CostEstimate(
            flops=2 * N * Hp * 2 * Wp * K * W + 8 * N * HW,
            transcendentals=N * HW,
            bytes_accessed=(3 * N * HW + 2 * Wp * K * W) * 4),
    )(avg.reshape(N, H, W), mx.reshape(N, H, W), s_mat)


def kernel(x, conv_weight):
    N, C, H, W = x.shape
    HW = H * W
    Hp, Wp = H + 2 * P, W + 2 * P

    s_mat = _band_matrix(conv_weight, W, Wp)
    lanes_ok = C % LANES == 0 and ((C // LANES) & (C // LANES - 1)) == 0
    if not lanes_ok:
        return _generic_path(x, s_mat).reshape(N, 1, H, W)

    nb = 4 if N % 4 == 0 else 1
    xt = jnp.transpose(x, (0, 2, 3, 1))
    out = pl.pallas_call(
        functools.partial(_fused_body, nb=nb, c=C, h=H, w=W, inv_c=1.0 / C),
        out_shape=jax.ShapeDtypeStruct((N, H, W), x.dtype),
        grid=(N // nb,),
        in_specs=[
            pl.BlockSpec((2 * Wp, K * W), lambda i: (0, 0)),
            pl.BlockSpec((nb, H, W, C), lambda i: (i, 0, 0, 0)),
        ],
        out_specs=pl.BlockSpec((nb, H, W), lambda i: (i, 0, 0)),
        scratch_shapes=[pltpu.VMEM((nb, Hp, 2 * Wp), jnp.float32)],
        compiler_params=pltpu.CompilerParams(
            dimension_semantics=("parallel",),
            vmem_limit_bytes=48 << 20),
        cost_estimate=pl.CostEstimate(
            flops=2 * N * C * HW + 2 * N * Hp * 2 * Wp * K * W,
            transcendentals=N * HW,
            bytes_accessed=(N * C * HW + N * HW) * 4),
    )(s_mat, xt)
    return out.reshape(N, 1, H, W)
